# baseline (speedup 1.0000x reference)
import functools

import jax
import jax.numpy as jnp
from jax import lax
from jax.experimental import pallas as pl
from jax.experimental.pallas import tpu as pltpu
from jax.experimental.pallas import tpu_sc as plsc

N = 10000
NPAD = 10016
DISCARD = 10008
IN_CH = 128
HID = 16
OUT_CH = 40
E = 320000
NC = 2
NS = 16
NW = NC * NS
CHUNK = 64
NCHUNK = 5120
CPT = NCHUNK // NW
EPAD = NCHUNK * CHUNK
NBUF = 8
NSUP = CPT // NBUF
ROWS_PT = 632
ROWS_LAST = NPAD - ROWS_PT * (NS - 1)
TROWS_PT = 632
TROWS_LAST = N - TROWS_PT * (NS - 1)

_MESH = plsc.VectorSubcoreMesh(
    core_axis_name="c", subcore_axis_name="s", num_cores=NC, num_subcores=NS
)
_SC_PARAMS = pltpu.CompilerParams(use_tc_tiling_on_sc=False)


def _wid():
    return lax.axis_index("s") * NC + lax.axis_index("c")


def _load_chunks(ehbm, which, ev, w):
    start = pl.multiple_of(CPT * w, 8)
    pltpu.sync_copy(ehbm.at[which, pl.ds(start, CPT)], ev)


@functools.partial(
    pl.kernel,
    out_type=jax.ShapeDtypeStruct((NC, NPAD), jnp.float32),
    mesh=_MESH,
    scratch_types=[
        pltpu.VMEM((CPT, CHUNK), jnp.int32),
        pltpu.VMEM((CHUNK,), jnp.float32),
        pltpu.VMEM_SHARED((NPAD,), jnp.float32),
        pltpu.SemaphoreType.DMA,
    ],
    compiler_params=_SC_PARAMS,
)
def _sc_degree(edges_hbm, zeros_hbm, out_hbm, dst_v, ones_v, acc, sem):
    c = lax.axis_index("c")
    s = lax.axis_index("s")
    _load_chunks(edges_hbm, 1, dst_v, _wid())
    for i in range(CHUNK // 16):
        ones_v[pl.ds(i * 16, 16)] = jnp.ones((16,), jnp.float32)

    @pl.when(s == 0)
    def _():
        pltpu.sync_copy(zeros_hbm, acc)

    plsc.subcore_barrier()

    def fire(j, carry):
        pltpu.async_copy(ones_v, acc.at[dst_v.at[j]], sem, add=True)
        return carry

    lax.fori_loop(0, CPT, fire, 0)

    def drain(j, carry):
        pltpu.make_async_copy(ones_v, acc.at[dst_v.at[0]], sem).wait()
        return carry

    lax.fori_loop(0, CPT, drain, 0)
    plsc.subcore_barrier()

    @pl.when(s == 0)
    def _():
        pltpu.sync_copy(acc, out_hbm.at[c])


@functools.partial(
    pl.kernel,
    out_type=jax.ShapeDtypeStruct((NC, NPAD, HID), jnp.float32),
    mesh=_MESH,
    scratch_types=[
        pltpu.VMEM((CPT, CHUNK), jnp.int32),
        pltpu.VMEM((CPT, CHUNK), jnp.int32),
        [pltpu.VMEM((CHUNK, HID), jnp.float32)] * NBUF,
        [pltpu.SemaphoreType.DMA] * NBUF,
        [pltpu.SemaphoreType.DMA] * NBUF,
        pltpu.VMEM_SHARED((NPAD, HID), jnp.float32),
        pltpu.VMEM_SHARED((N, HID), jnp.float32),
    ],
    compiler_params=_SC_PARAMS,
)
def _sc_propagate(table_hbm, edges_hbm, zeros_hbm, out_hbm,
                  src_v, dst_v, rows, gsem, ssem, acc, table):
    c = lax.axis_index("c")
    s = lax.axis_index("s")
    w = _wid()
    _load_chunks(edges_hbm, 0, src_v, w)
    _load_chunks(edges_hbm, 1, dst_v, w)

    abase = pl.multiple_of(s * ROWS_PT, 8)
    tbase = pl.multiple_of(s * TROWS_PT, 8)

    @pl.when(s < NS - 1)
    def _():
        pltpu.sync_copy(zeros_hbm.at[pl.ds(abase, ROWS_PT)],
                        acc.at[pl.ds(abase, ROWS_PT)])
        pltpu.sync_copy(table_hbm.at[pl.ds(tbase, TROWS_PT)],
                        table.at[pl.ds(tbase, TROWS_PT)])

    @pl.when(s == NS - 1)
    def _():
        pltpu.sync_copy(
            zeros_hbm.at[pl.ds((NS - 1) * ROWS_PT, ROWS_LAST)],
            acc.at[pl.ds((NS - 1) * ROWS_PT, ROWS_LAST)])
        pltpu.sync_copy(
            table_hbm.at[pl.ds((NS - 1) * TROWS_PT, TROWS_LAST)],
            table.at[pl.ds((NS - 1) * TROWS_PT, TROWS_LAST)])

    plsc.subcore_barrier()

    def gather(k, i):
        pltpu.async_copy(table.at[src_v.at[k]], rows[i], gsem[i])

    def gather_done(i):
        pltpu.make_async_copy(table.at[src_v.at[0]], rows[i],
                              gsem[i]).wait()

    def scatter(k, i):
        pltpu.async_copy(rows[i], acc.at[dst_v.at[k]], ssem[i], add=True)

    def scatter_done(i):
        pltpu.make_async_copy(rows[i], acc.at[dst_v.at[0]], ssem[i]).wait()

    for i in range(NBUF):
        gather(i, i)

    def superchunk(t, carry):
        base = NBUF * t
        for i in range(NBUF):
            k = base + i
            gather_done(i)
            scatter(k, i)

            @pl.when(t < NSUP - 1)
            def _():
                scatter_done(i)
                gather(k + NBUF, i)
        return carry

    lax.fori_loop(0, NSUP, superchunk, 0)
    for i in range(NBUF):
        scatter_done(i)
    plsc.subcore_barrier()

    @pl.when(s < NS - 1)
    def _():
        pltpu.sync_copy(acc.at[pl.ds(abase, ROWS_PT)],
                        out_hbm.at[c, pl.ds(abase, ROWS_PT)])

    @pl.when(s == NS - 1)
    def _():
        pltpu.sync_copy(
            acc.at[pl.ds((NS - 1) * ROWS_PT, ROWS_LAST)],
            out_hbm.at[c, pl.ds((NS - 1) * ROWS_PT, ROWS_LAST)])


def _tc_mm_body(x_ref, w1_ref, xw_ref):
    xw_ref[...] = jnp.dot(x_ref[...], w1_ref[...],
                          preferred_element_type=jnp.float32)


def _tc1_body(xw_ref, degp_ref, xs1_ref, dis_ref):
    deg = sum(degp_ref[i, :N] for i in range(NC)) + 1.0
    dis = lax.rsqrt(deg)
    dis_ref[...] = dis
    xs1_ref[...] = xw_ref[...] * dis


def _tc2_body(y1p_ref, xs1_ref, dis_ref, b1_ref, xs2_ref):
    y = sum(y1p_ref[i, :N] for i in range(NC)) + xs1_ref[...]
    h = jnp.maximum(dis_ref[...] * y + b1_ref[...], 0.0)
    xs2_ref[...] = h * dis_ref[...]


def _tc3_body(y2p_ref, xs2_ref, dis_ref, w2_ref, b2_ref, out_ref):
    g = dis_ref[...] * (sum(y2p_ref[i, :N] for i in range(NC)) + xs2_ref[...])
    o = jnp.dot(g, w2_ref[...], preferred_element_type=jnp.float32) + b2_ref[...]
    m = jnp.max(o, axis=1, keepdims=True)
    e = o - m
    lse = jnp.log(jnp.sum(jnp.exp(e), axis=1, keepdims=True))
    out_ref[...] = e - lse


def kernel(x, edge_index, W1, b1, W2, b2):
    e32 = edge_index.astype(jnp.int32)
    padcols = jnp.stack([
        jnp.zeros((EPAD - E,), jnp.int32),
        jnp.full((EPAD - E,), DISCARD, jnp.int32),
    ])
    edges = jnp.concatenate([e32, padcols], axis=1).reshape(2, NCHUNK, CHUNK)
    z1 = jnp.zeros((NPAD,), jnp.float32)
    z16 = jnp.zeros((NPAD, HID), jnp.float32)

    degp = _sc_degree(edges, z1)
    xw = pl.pallas_call(
        _tc_mm_body,
        out_shape=jax.ShapeDtypeStruct((N, HID), jnp.float32),
    )(x, W1)

    xs1, dis = pl.pallas_call(
        _tc1_body,
        out_shape=(
            jax.ShapeDtypeStruct((N, HID), jnp.float32),
            jax.ShapeDtypeStruct((N, 1), jnp.float32),
        ),
    )(xw, degp.reshape(NC, NPAD, 1))

    y1p = _sc_propagate(xs1, edges, z16)

    xs2 = pl.pallas_call(
        _tc2_body,
        out_shape=jax.ShapeDtypeStruct((N, HID), jnp.float32),
    )(y1p, xs1, dis, b1.reshape(1, HID))

    y2p = _sc_propagate(xs2, edges, z16)

    out = pl.pallas_call(
        _tc3_body,
        out_shape=jax.ShapeDtypeStruct((N, OUT_CH), jnp.float32),
    )(y2p, xs2, dis, W2, b2.reshape(1, OUT_CH))
    return out


# speedup vs baseline: 1.0950x; 1.0950x over previous
import functools

import jax
import jax.numpy as jnp
from jax import lax
from jax.experimental import pallas as pl
from jax.experimental.pallas import tpu as pltpu
from jax.experimental.pallas import tpu_sc as plsc

N = 10000
NPAD = 10016
DISCARD = 10008
IN_CH = 128
HID = 16
HH = 8
OUT_CH = 40
E = 320000
NC = 2
NS = 16
NW = NC * NS
CHUNK = 128
NCHUNK = 2560
EPAD = NCHUNK * CHUNK
CPT = NCHUNK // NW
CPTM = NCHUNK // NS
NBUF = 8
ROWS_PT = 632
ROWS_LAST = NPAD - ROWS_PT * (NS - 1)
TROWS_PT = 632
TROWS_LAST = N - TROWS_PT * (NS - 1)

_MESH = plsc.VectorSubcoreMesh(
    core_axis_name="c", subcore_axis_name="s", num_cores=NC, num_subcores=NS
)
_SC_PARAMS = pltpu.CompilerParams(use_tc_tiling_on_sc=False)
_SC_PARAMS_NL = pltpu.CompilerParams(use_tc_tiling_on_sc=False,
                                     needs_layout_passes=False)


@functools.partial(
    pl.kernel,
    out_type=jax.ShapeDtypeStruct((NC, NPAD), jnp.float32),
    mesh=_MESH,
    scratch_types=[
        pltpu.VMEM((CPT, CHUNK), jnp.int32),
        pltpu.VMEM((CHUNK,), jnp.float32),
        pltpu.VMEM_SHARED((NPAD,), jnp.float32),
        pltpu.SemaphoreType.DMA,
    ],
    compiler_params=_SC_PARAMS,
)
def _sc_degree(edges_hbm, zeros_hbm, out_hbm, dst_v, ones_v, acc, sem):
    c = lax.axis_index("c")
    s = lax.axis_index("s")
    w = s * NC + c
    start = pl.multiple_of(CPT * w, 8)
    pltpu.sync_copy(edges_hbm.at[1, pl.ds(start, CPT)], dst_v)
    for i in range(CHUNK // 16):
        ones_v[pl.ds(i * 16, 16)] = jnp.ones((16,), jnp.float32)

    @pl.when(s == 0)
    def _():
        pltpu.sync_copy(zeros_hbm, acc)

    plsc.subcore_barrier()

    def fire(j, carry):
        pltpu.async_copy(ones_v, acc.at[dst_v.at[j]], sem, add=True)
        return carry

    lax.fori_loop(0, CPT, fire, 0)

    def drain(j, carry):
        pltpu.make_async_copy(ones_v, acc.at[dst_v.at[0]], sem).wait()
        return carry

    lax.fori_loop(0, CPT, drain, 0)
    plsc.subcore_barrier()

    @pl.when(s == 0)
    def _():
        pltpu.sync_copy(acc, out_hbm.at[c])


@functools.partial(
    pl.kernel,
    out_type=jax.ShapeDtypeStruct((N, HID), jnp.float32),
    mesh=_MESH,
    scratch_types=[
        pltpu.VMEM((CPTM, CHUNK), jnp.int32),
        pltpu.VMEM((CPTM, CHUNK), jnp.int32),
        [pltpu.VMEM((CHUNK, HH), jnp.float32)] * NBUF,
        [pltpu.SemaphoreType.DMA] * NBUF,
        [pltpu.SemaphoreType.DMA] * NBUF,
        pltpu.VMEM((TROWS_PT, HH), jnp.float32),
        pltpu.VMEM((TROWS_PT, HH), jnp.float32),
        pltpu.VMEM((TROWS_PT, HH), jnp.float32),
        pltpu.VMEM((TROWS_PT, HH), jnp.float32),
        pltpu.VMEM((TROWS_PT, HH), jnp.float32),
        pltpu.VMEM((16,), jnp.float32),
        pltpu.VMEM_SHARED((N, HH), jnp.float32),
        pltpu.VMEM_SHARED((N, HH), jnp.float32),
        pltpu.VMEM_SHARED((NPAD, HH), jnp.float32),
    ],
    compiler_params=_SC_PARAMS_NL,
)
def _sc_mega(xs1_hbm, edges_hbm, dis_hbm, b1p_hbm, zeros8_hbm, out_hbm,
             src_v, dst_v, rows, gsem, ssem,
             ybuf, xbuf, dbuf, t2buf, gbuf, bbuf, table1, table2, acc):
    c = lax.axis_index("c")
    s = lax.axis_index("s")
    start = pl.multiple_of(CPTM * s, 8)
    pltpu.sync_copy(edges_hbm.at[0, pl.ds(start, CPTM)], src_v)
    pltpu.sync_copy(edges_hbm.at[1, pl.ds(start, CPTM)], dst_v)
    pltpu.sync_copy(b1p_hbm.at[c], bbuf)

    nbase = pl.multiple_of(s * TROWS_PT, 8)
    abase = pl.multiple_of(s * ROWS_PT, 8)
    last = NS - 1

    def stage(nrows, arows):
        for cc in range(NC):
            @pl.when(c == cc)
            def _():
                pltpu.sync_copy(
                    xs1_hbm.at[pl.ds(nbase, nrows), pl.ds(cc * HH, HH)],
                    table1.at[pl.ds(nbase, nrows)])
        pltpu.sync_copy(dis_hbm.at[pl.ds(nbase, nrows), pl.ds(0, HH)],
                        dbuf.at[pl.ds(0, nrows)])
        pltpu.sync_copy(zeros8_hbm.at[pl.ds(abase, arows)],
                        acc.at[pl.ds(abase, arows)])

    @pl.when(s < last)
    def _():
        stage(TROWS_PT, ROWS_PT)

    @pl.when(s == last)
    def _():
        stage(TROWS_LAST, ROWS_LAST)

    plsc.subcore_barrier()

    def run_pass(table):
        def gather(k, i):
            pltpu.async_copy(table.at[src_v.at[k]], rows[i], gsem[i])

        def gather_done(i):
            pltpu.make_async_copy(table.at[src_v.at[0]], rows[i],
                                  gsem[i]).wait()

        def scatter(k, i):
            pltpu.async_copy(rows[i], acc.at[dst_v.at[k]], ssem[i], add=True)

        def scatter_done(i):
            pltpu.make_async_copy(rows[i], acc.at[dst_v.at[0]],
                                  ssem[i]).wait()

        for i in range(NBUF):
            gather(i, i)

        nsup = CPTM // NBUF

        def superchunk(t, carry):
            base = NBUF * t
            for i in range(NBUF):
                k = base + i
                gather_done(i)
                scatter(k, i)

                @pl.when(t < nsup - 1)
                def _():
                    scatter_done(i)
                    gather(k + NBUF, i)
            return carry

        lax.fori_loop(0, nsup, superchunk, 0)
        for i in range(NBUF):
            scatter_done(i)

    run_pass(table1)
    plsc.subcore_barrier()

    i16 = lax.iota(jnp.int32, 16)
    r0 = lax.shift_right_logical(i16, 3)
    cidx = lax.bitwise_and(i16, jnp.full((16,), 7, jnp.int32))
    b1v = bbuf[...]
    npairs = jnp.where(s == last, TROWS_LAST // 2, TROWS_PT // 2)

    def load_rows(nrows):
        pltpu.sync_copy(acc.at[pl.ds(nbase, nrows)], ybuf.at[pl.ds(0, nrows)])
        pltpu.sync_copy(table1.at[pl.ds(nbase, nrows)],
                        xbuf.at[pl.ds(0, nrows)])

    @pl.when(s < last)
    def _():
        load_rows(TROWS_PT)

    @pl.when(s == last)
    def _():
        load_rows(TROWS_LAST)

    def midrow(k, carry):
        r = r0 + 2 * k
        yv = plsc.load_gather(ybuf, [r, cidx])
        xv = plsc.load_gather(xbuf, [r, cidx])
        dv = plsc.load_gather(dbuf, [r, cidx])
        hv = jnp.maximum(dv * (yv + xv) + b1v, 0.0)
        plsc.store_scatter(t2buf, [r, cidx], hv * dv)
        return carry

    lax.fori_loop(0, npairs, midrow, 0)

    def put_t2(nrows, arows):
        pltpu.sync_copy(t2buf.at[pl.ds(0, nrows)],
                        table2.at[pl.ds(nbase, nrows)])
        pltpu.sync_copy(zeros8_hbm.at[pl.ds(abase, arows)],
                        acc.at[pl.ds(abase, arows)])

    @pl.when(s < last)
    def _():
        put_t2(TROWS_PT, ROWS_PT)

    @pl.when(s == last)
    def _():
        put_t2(TROWS_LAST, ROWS_LAST)

    plsc.subcore_barrier()
    run_pass(table2)
    plsc.subcore_barrier()

    @pl.when(s < last)
    def _():
        pltpu.sync_copy(acc.at[pl.ds(nbase, TROWS_PT)],
                        ybuf.at[pl.ds(0, TROWS_PT)])

    @pl.when(s == last)
    def _():
        pltpu.sync_copy(acc.at[pl.ds(nbase, TROWS_LAST)],
                        ybuf.at[pl.ds(0, TROWS_LAST)])

    def finrow(k, carry):
        r = r0 + 2 * k
        yv = plsc.load_gather(ybuf, [r, cidx])
        xv = plsc.load_gather(t2buf, [r, cidx])
        dv = plsc.load_gather(dbuf, [r, cidx])
        plsc.store_scatter(gbuf, [r, cidx], dv * (yv + xv))
        return carry

    lax.fori_loop(0, npairs, finrow, 0)

    def put_g(nrows):
        for cc in range(NC):
            @pl.when(c == cc)
            def _():
                pltpu.sync_copy(
                    gbuf.at[pl.ds(0, nrows)],
                    out_hbm.at[pl.ds(nbase, nrows), pl.ds(cc * HH, HH)])

    @pl.when(s < last)
    def _():
        put_g(TROWS_PT)

    @pl.when(s == last)
    def _():
        put_g(TROWS_LAST)


def _tc_mm_body(x_ref, w1_ref, xw_ref):
    xw_ref[...] = jnp.dot(x_ref[...], w1_ref[...],
                          preferred_element_type=jnp.float32)


def _tc1_body(xw_ref, degp_ref, xs1_ref, dis_ref):
    deg = sum(degp_ref[i, :N] for i in range(NC)) + 1.0
    dis = lax.rsqrt(deg)
    dis_ref[...] = jnp.broadcast_to(dis, (N, HID))
    xs1_ref[...] = xw_ref[...] * dis


def _tc3_body(g_ref, w2_ref, b2_ref, out_ref):
    o = jnp.dot(g_ref[...], w2_ref[...],
                preferred_element_type=jnp.float32) + b2_ref[...]
    m = jnp.max(o, axis=1, keepdims=True)
    e = o - m
    lse = jnp.log(jnp.sum(jnp.exp(e), axis=1, keepdims=True))
    out_ref[...] = e - lse


def kernel(x, edge_index, W1, b1, W2, b2):
    e32 = edge_index.astype(jnp.int32)
    padcols = jnp.stack([
        jnp.zeros((EPAD - E,), jnp.int32),
        jnp.full((EPAD - E,), DISCARD, jnp.int32),
    ])
    edges = jnp.concatenate([e32, padcols], axis=1).reshape(2, NCHUNK, CHUNK)
    z1 = jnp.zeros((NPAD,), jnp.float32)
    z8 = jnp.zeros((NPAD, HH), jnp.float32)
    b1p = jnp.stack([jnp.tile(b1[:HH], 2), jnp.tile(b1[HH:], 2)])

    degp = _sc_degree(edges, z1)
    xw = pl.pallas_call(
        _tc_mm_body,
        out_shape=jax.ShapeDtypeStruct((N, HID), jnp.float32),
    )(x, W1)

    xs1, dis = pl.pallas_call(
        _tc1_body,
        out_shape=(
            jax.ShapeDtypeStruct((N, HID), jnp.float32),
            jax.ShapeDtypeStruct((N, HID), jnp.float32),
        ),
    )(xw, degp.reshape(NC, NPAD, 1))

    g = _sc_mega(xs1, edges, dis, b1p, z8)

    out = pl.pallas_call(
        _tc3_body,
        out_shape=jax.ShapeDtypeStruct((N, OUT_CH), jnp.float32),
    )(g, W2, b2.reshape(1, OUT_CH))
    return out


# speedup vs baseline: 1.2483x; 1.1401x over previous
import functools

import jax
import jax.numpy as jnp
from jax import lax
from jax.experimental import pallas as pl
from jax.experimental.pallas import tpu as pltpu
from jax.experimental.pallas import tpu_sc as plsc

N = 10000
NPAD = 10016
DISCARD = 10008
IN_CH = 128
HID = 16
HH = 8
OUT_CH = 40
E = 320000
NC = 2
NS = 16
NW = NC * NS
CHUNK = 128
NCHUNK = 2560
EPAD = NCHUNK * CHUNK
CPT = NCHUNK // NW
CPTM = NCHUNK // NS
NBUF = 8
ROWS_PT = 632
ROWS_LAST = NPAD - ROWS_PT * (NS - 1)
TROWS_PT = 632
TROWS_LAST = N - TROWS_PT * (NS - 1)

_MESH = plsc.VectorSubcoreMesh(
    core_axis_name="c", subcore_axis_name="s", num_cores=NC, num_subcores=NS
)
_SC_PARAMS = pltpu.CompilerParams(use_tc_tiling_on_sc=False)
_SC_PARAMS_NL = pltpu.CompilerParams(use_tc_tiling_on_sc=False,
                                     needs_layout_passes=False)


@functools.partial(
    pl.kernel,
    out_type=jax.ShapeDtypeStruct((N, HID), jnp.float32),
    mesh=_MESH,
    scratch_types=[
        pltpu.VMEM((CPTM, CHUNK), jnp.int32),
        pltpu.VMEM((CPTM, CHUNK), jnp.int32),
        [pltpu.VMEM((CHUNK, HH), jnp.float32)] * NBUF,
        [pltpu.SemaphoreType.DMA] * NBUF,
        [pltpu.SemaphoreType.DMA] * NBUF,
        pltpu.VMEM((TROWS_PT, HH), jnp.float32),
        pltpu.VMEM((TROWS_PT, HH), jnp.float32),
        pltpu.VMEM((TROWS_PT, HH), jnp.float32),
        pltpu.VMEM((TROWS_PT, HH), jnp.float32),
        pltpu.VMEM((16,), jnp.float32),
        pltpu.VMEM((CHUNK,), jnp.float32),
        pltpu.VMEM((640,), jnp.float32),
        pltpu.SemaphoreType.DMA,
        pltpu.VMEM_SHARED((N, HH), jnp.float32),
        pltpu.VMEM_SHARED((N, HH), jnp.float32),
        pltpu.VMEM_SHARED((NPAD, HH), jnp.float32),
        pltpu.VMEM_SHARED((NPAD,), jnp.float32),
    ],
    compiler_params=_SC_PARAMS_NL,
)
def _sc_mega(xw_hbm, edges_hbm, b1p_hbm, zeros8_hbm, zeros1_hbm, out_hbm,
             src_v, dst_v, rows, gsem, ssem,
             ybuf, xbuf, t2buf, gbuf, bbuf, ones_v, disb, dsem,
             table1, table2, acc, dacc):
    c = lax.axis_index("c")
    s = lax.axis_index("s")
    start = pl.multiple_of(CPTM * s, 8)
    pltpu.sync_copy(edges_hbm.at[0, pl.ds(start, CPTM)], src_v)
    pltpu.sync_copy(edges_hbm.at[1, pl.ds(start, CPTM)], dst_v)
    pltpu.sync_copy(b1p_hbm.at[c], bbuf)
    for i in range(CHUNK // 16):
        ones_v[pl.ds(i * 16, 16)] = jnp.ones((16,), jnp.float32)

    nbase = pl.multiple_of(s * TROWS_PT, 8)
    abase = pl.multiple_of(s * ROWS_PT, 8)
    last = NS - 1

    def stage(nrows, arows):
        for cc in range(NC):
            @pl.when(c == cc)
            def _():
                pltpu.sync_copy(
                    xw_hbm.at[pl.ds(nbase, nrows), pl.ds(cc * HH, HH)],
                    xbuf.at[pl.ds(0, nrows)])
        pltpu.sync_copy(zeros8_hbm.at[pl.ds(abase, arows)],
                        acc.at[pl.ds(abase, arows)])
        pltpu.sync_copy(zeros1_hbm.at[pl.ds(abase, arows)],
                        dacc.at[pl.ds(abase, arows)])

    @pl.when(s < last)
    def _():
        stage(TROWS_PT, ROWS_PT)

    @pl.when(s == last)
    def _():
        stage(TROWS_LAST, ROWS_LAST)

    plsc.subcore_barrier()

    def dfire(j, carry):
        pltpu.async_copy(ones_v, dacc.at[dst_v.at[j]], dsem, add=True)
        return carry

    lax.fori_loop(0, CPTM, dfire, 0)

    def ddrain(j, carry):
        pltpu.make_async_copy(ones_v, dacc.at[dst_v.at[0]], dsem).wait()
        return carry

    lax.fori_loop(0, CPTM, ddrain, 0)
    plsc.subcore_barrier()

    @pl.when(s < last)
    def _():
        pltpu.sync_copy(dacc.at[pl.ds(nbase, TROWS_PT)],
                        disb.at[pl.ds(0, TROWS_PT)])

    @pl.when(s == last)
    def _():
        pltpu.sync_copy(dacc.at[pl.ds(nbase, TROWS_LAST)],
                        disb.at[pl.ds(0, TROWS_LAST)])

    magic = jnp.full((16,), 0x5F3759DF, jnp.int32)

    def newton(k, carry):
        xv = disb[pl.ds(k * 16, 16)] + 1.0
        iv = magic - lax.shift_right_logical(plsc.bitcast(xv, jnp.int32), 1)
        yv = plsc.bitcast(iv, jnp.float32)
        for _ in range(3):
            yv = yv * (1.5 - 0.5 * xv * yv * yv)
        disb[pl.ds(k * 16, 16)] = yv
        return carry

    lax.fori_loop(0, 640 // 16, newton, 0)

    i16 = lax.iota(jnp.int32, 16)
    r0 = lax.shift_right_logical(i16, 3)
    cidx = lax.bitwise_and(i16, jnp.full((16,), 7, jnp.int32))
    npairs = jnp.where(s == last, TROWS_LAST // 2, TROWS_PT // 2)

    def prerow(k, carry):
        r = r0 + 2 * k
        dv = plsc.load_gather(disb, [r])
        tv = plsc.load_gather(xbuf, [r, cidx]) * dv
        plsc.store_scatter(gbuf, [r, cidx], tv)
        return carry

    lax.fori_loop(0, npairs, prerow, 0)

    @pl.when(s < last)
    def _():
        pltpu.sync_copy(gbuf.at[pl.ds(0, TROWS_PT)],
                        table1.at[pl.ds(nbase, TROWS_PT)])

    @pl.when(s == last)
    def _():
        pltpu.sync_copy(gbuf.at[pl.ds(0, TROWS_LAST)],
                        table1.at[pl.ds(nbase, TROWS_LAST)])

    plsc.subcore_barrier()

    def run_pass(table):
        def gather(k, i):
            pltpu.async_copy(table.at[src_v.at[k]], rows[i], gsem[i])

        def gather_done(i):
            pltpu.make_async_copy(table.at[src_v.at[0]], rows[i],
                                  gsem[i]).wait()

        def scatter(k, i):
            pltpu.async_copy(rows[i], acc.at[dst_v.at[k]], ssem[i], add=True)

        def scatter_done(i):
            pltpu.make_async_copy(rows[i], acc.at[dst_v.at[0]],
                                  ssem[i]).wait()

        for i in range(NBUF):
            gather(i, i)

        nsup = CPTM // NBUF

        def superchunk(t, carry):
            base = NBUF * t
            for i in range(NBUF):
                k = base + i
                gather_done(i)
                scatter(k, i)

                @pl.when(t < nsup - 1)
                def _():
                    scatter_done(i)
                    gather(k + NBUF, i)
            return carry

        lax.fori_loop(0, nsup, superchunk, 0)
        for i in range(NBUF):
            scatter_done(i)

    run_pass(table1)
    plsc.subcore_barrier()

    b1v = bbuf[...]

    def load_rows(nrows):
        pltpu.sync_copy(acc.at[pl.ds(nbase, nrows)], ybuf.at[pl.ds(0, nrows)])
        pltpu.sync_copy(table1.at[pl.ds(nbase, nrows)],
                        xbuf.at[pl.ds(0, nrows)])

    @pl.when(s < last)
    def _():
        load_rows(TROWS_PT)

    @pl.when(s == last)
    def _():
        load_rows(TROWS_LAST)

    def midrow(k, carry):
        r = r0 + 2 * k
        yv = plsc.load_gather(ybuf, [r, cidx])
        xv = plsc.load_gather(xbuf, [r, cidx])
        dv = plsc.load_gather(disb, [r])
        hv = jnp.maximum(dv * (yv + xv) + b1v, 0.0)
        plsc.store_scatter(t2buf, [r, cidx], hv * dv)
        return carry

    lax.fori_loop(0, npairs, midrow, 0)

    def put_t2(nrows, arows):
        pltpu.sync_copy(t2buf.at[pl.ds(0, nrows)],
                        table2.at[pl.ds(nbase, nrows)])
        pltpu.sync_copy(zeros8_hbm.at[pl.ds(abase, arows)],
                        acc.at[pl.ds(abase, arows)])

    @pl.when(s < last)
    def _():
        put_t2(TROWS_PT, ROWS_PT)

    @pl.when(s == last)
    def _():
        put_t2(TROWS_LAST, ROWS_LAST)

    plsc.subcore_barrier()
    run_pass(table2)
    plsc.subcore_barrier()

    @pl.when(s < last)
    def _():
        pltpu.sync_copy(acc.at[pl.ds(nbase, TROWS_PT)],
                        ybuf.at[pl.ds(0, TROWS_PT)])

    @pl.when(s == last)
    def _():
        pltpu.sync_copy(acc.at[pl.ds(nbase, TROWS_LAST)],
                        ybuf.at[pl.ds(0, TROWS_LAST)])

    def finrow(k, carry):
        r = r0 + 2 * k
        yv = plsc.load_gather(ybuf, [r, cidx])
        xv = plsc.load_gather(t2buf, [r, cidx])
        dv = plsc.load_gather(disb, [r])
        plsc.store_scatter(gbuf, [r, cidx], dv * (yv + xv))
        return carry

    lax.fori_loop(0, npairs, finrow, 0)

    def put_g(nrows):
        for cc in range(NC):
            @pl.when(c == cc)
            def _():
                pltpu.sync_copy(
                    gbuf.at[pl.ds(0, nrows)],
                    out_hbm.at[pl.ds(nbase, nrows), pl.ds(cc * HH, HH)])

    @pl.when(s < last)
    def _():
        put_g(TROWS_PT)

    @pl.when(s == last)
    def _():
        put_g(TROWS_LAST)


def _tc_mm_body(x_ref, w1_ref, xw_ref):
    xw_ref[...] = jnp.dot(x_ref[...], w1_ref[...],
                          preferred_element_type=jnp.float32)


def _tc3_body(g_ref, w2_ref, b2_ref, out_ref):
    o = jnp.dot(g_ref[...], w2_ref[...],
                preferred_element_type=jnp.float32) + b2_ref[...]
    m = jnp.max(o, axis=1, keepdims=True)
    e = o - m
    lse = jnp.log(jnp.sum(jnp.exp(e), axis=1, keepdims=True))
    out_ref[...] = e - lse


def kernel(x, edge_index, W1, b1, W2, b2):
    e32 = edge_index.astype(jnp.int32)
    padcols = jnp.stack([
        jnp.zeros((EPAD - E,), jnp.int32),
        jnp.full((EPAD - E,), DISCARD, jnp.int32),
    ])
    edges = jnp.concatenate([e32, padcols], axis=1).reshape(2, NCHUNK, CHUNK)
    z1 = jnp.zeros((NPAD,), jnp.float32)
    z8 = jnp.zeros((NPAD, HH), jnp.float32)
    b1p = jnp.stack([jnp.tile(b1[:HH], 2), jnp.tile(b1[HH:], 2)])

    xw = pl.pallas_call(
        _tc_mm_body,
        out_shape=jax.ShapeDtypeStruct((N, HID), jnp.float32),
    )(x, W1)

    g = _sc_mega(xw, edges, b1p, z8, z1)

    out = pl.pallas_call(
        _tc3_body,
        out_shape=jax.ShapeDtypeStruct((N, OUT_CH), jnp.float32),
    )(g, W2, b2.reshape(1, OUT_CH))
    return out


# speedup vs baseline: 1.2486x; 1.0002x over previous
import functools

import jax
import jax.numpy as jnp
from jax import lax
from jax.experimental import pallas as pl
from jax.experimental.pallas import tpu as pltpu
from jax.experimental.pallas import tpu_sc as plsc

N = 10000
NPAD = 10016
DISCARD = 10008
IN_CH = 128
HID = 16
HH = 8
OUT_CH = 40
E = 320000
NC = 2
NS = 16
NW = NC * NS
CHUNK = 128
NCHUNK = 2560
EPAD = NCHUNK * CHUNK
CPTM = NCHUNK // NS
NBUF = 8
ROWS_PT = 632
ROWS_LAST = NPAD - ROWS_PT * (NS - 1)
TROWS_PT = 632
TROWS_LAST = N - TROWS_PT * (NS - 1)

_MESH = plsc.VectorSubcoreMesh(
    core_axis_name="c", subcore_axis_name="s", num_cores=NC, num_subcores=NS
)
_SC_PARAMS_NL = pltpu.CompilerParams(use_tc_tiling_on_sc=False,
                                     needs_layout_passes=False)


@functools.partial(
    pl.kernel,
    out_type=jax.ShapeDtypeStruct((N, HID), jnp.float32),
    mesh=_MESH,
    scratch_types=[
        pltpu.VMEM((CPTM, CHUNK), jnp.int32),
        pltpu.VMEM((CPTM, CHUNK), jnp.int32),
        [pltpu.VMEM((CHUNK, HH), jnp.float32)] * NBUF,
        [pltpu.SemaphoreType.DMA] * NBUF,
        [pltpu.SemaphoreType.DMA] * NBUF,
        pltpu.VMEM((TROWS_PT, HH), jnp.float32),
        pltpu.VMEM((TROWS_PT, HH), jnp.float32),
        pltpu.VMEM((TROWS_PT, HH), jnp.float32),
        pltpu.VMEM((TROWS_PT, HH), jnp.float32),
        pltpu.VMEM((16,), jnp.float32),
        pltpu.VMEM((CHUNK,), jnp.float32),
        pltpu.VMEM((640,), jnp.float32),
        pltpu.SemaphoreType.DMA,
        pltpu.VMEM_SHARED((N, HH), jnp.float32),
        pltpu.VMEM_SHARED((N, HH), jnp.float32),
        pltpu.VMEM_SHARED((NPAD, HH), jnp.float32),
        pltpu.VMEM_SHARED((NPAD,), jnp.float32),
    ],
    compiler_params=_SC_PARAMS_NL,
)
def _sc_mega(xw_hbm, edges_hbm, b1p_hbm, zeros8_hbm, zeros1_hbm, out_hbm,
             src_v, dst_v, rows, gsem, ssem,
             ybuf, xbuf, t2buf, gbuf, bbuf, ones_v, disb, dsem,
             table1, table2, acc, dacc):
    c = lax.axis_index("c")
    s = lax.axis_index("s")
    start = pl.multiple_of(CPTM * s, 8)
    pltpu.sync_copy(edges_hbm.at[0, pl.ds(start, CPTM)], src_v)
    pltpu.sync_copy(edges_hbm.at[1, pl.ds(start, CPTM)], dst_v)
    pltpu.sync_copy(b1p_hbm.at[c], bbuf)
    for i in range(CHUNK // 16):
        ones_v[pl.ds(i * 16, 16)] = jnp.ones((16,), jnp.float32)

    nbase = pl.multiple_of(s * TROWS_PT, 8)
    abase = pl.multiple_of(s * ROWS_PT, 8)
    last = NS - 1

    def stage(nrows, arows):
        for cc in range(NC):
            @pl.when(c == cc)
            def _():
                pltpu.sync_copy(
                    xw_hbm.at[pl.ds(nbase, nrows), pl.ds(cc * HH, HH)],
                    xbuf.at[pl.ds(0, nrows)])
        pltpu.sync_copy(zeros8_hbm.at[pl.ds(abase, arows)],
                        acc.at[pl.ds(abase, arows)])
        pltpu.sync_copy(zeros1_hbm.at[pl.ds(abase, arows)],
                        dacc.at[pl.ds(abase, arows)])

    @pl.when(s < last)
    def _():
        stage(TROWS_PT, ROWS_PT)

    @pl.when(s == last)
    def _():
        stage(TROWS_LAST, ROWS_LAST)

    plsc.subcore_barrier()

    def dfire(j, carry):
        pltpu.async_copy(ones_v, dacc.at[dst_v.at[j]], dsem, add=True)
        return carry

    lax.fori_loop(0, CPTM, dfire, 0)

    def ddrain(j, carry):
        pltpu.make_async_copy(ones_v, dacc.at[dst_v.at[0]], dsem).wait()
        return carry

    lax.fori_loop(0, CPTM, ddrain, 0)
    plsc.subcore_barrier()

    @pl.when(s < last)
    def _():
        pltpu.sync_copy(dacc.at[pl.ds(nbase, TROWS_PT)],
                        disb.at[pl.ds(0, TROWS_PT)])

    @pl.when(s == last)
    def _():
        pltpu.sync_copy(dacc.at[pl.ds(nbase, TROWS_LAST)],
                        disb.at[pl.ds(0, TROWS_LAST)])

    magic = jnp.full((16,), 0x5F3759DF, jnp.int32)

    def newton(k, carry):
        xv = disb[pl.ds(k * 16, 16)] + 1.0
        iv = magic - lax.shift_right_logical(plsc.bitcast(xv, jnp.int32), 1)
        yv = plsc.bitcast(iv, jnp.float32)
        for _ in range(3):
            yv = yv * (1.5 - 0.5 * xv * yv * yv)
        disb[pl.ds(k * 16, 16)] = yv
        return carry

    lax.fori_loop(0, 640 // 16, newton, 0)

    i16 = lax.iota(jnp.int32, 16)
    r0 = lax.shift_right_logical(i16, 3)
    cidx = lax.bitwise_and(i16, jnp.full((16,), 7, jnp.int32))
    npairs = jnp.where(s == last, TROWS_LAST // 2, TROWS_PT // 2)

    def prerow(k, carry):
        r = r0 + 2 * k
        dv = plsc.load_gather(disb, [r])
        tv = plsc.load_gather(xbuf, [r, cidx]) * dv
        plsc.store_scatter(gbuf, [r, cidx], tv)
        return carry

    lax.fori_loop(0, npairs, prerow, 0)

    @pl.when(s < last)
    def _():
        pltpu.sync_copy(gbuf.at[pl.ds(0, TROWS_PT)],
                        table1.at[pl.ds(nbase, TROWS_PT)])

    @pl.when(s == last)
    def _():
        pltpu.sync_copy(gbuf.at[pl.ds(0, TROWS_LAST)],
                        table1.at[pl.ds(nbase, TROWS_LAST)])

    plsc.subcore_barrier()

    def run_pass(table):
        def gather(k, i):
            pltpu.async_copy(table.at[src_v.at[k]], rows[i], gsem[i])

        def gather_done(i):
            pltpu.make_async_copy(table.at[src_v.at[0]], rows[i],
                                  gsem[i]).wait()

        def scatter(k, i):
            pltpu.async_copy(rows[i], acc.at[dst_v.at[k]], ssem[i], add=True)

        def scatter_done(i):
            pltpu.make_async_copy(rows[i], acc.at[dst_v.at[0]],
                                  ssem[i]).wait()

        for i in range(NBUF):
            gather(i, i)

        nsup = CPTM // NBUF

        def superchunk(t, carry):
            base = NBUF * t
            for i in range(NBUF):
                k = base + i
                gather_done(i)
                scatter(k, i)

                @pl.when(t < nsup - 1)
                def _():
                    scatter_done(i)
                    gather(k + NBUF, i)
            return carry

        lax.fori_loop(0, nsup, superchunk, 0)
        for i in range(NBUF):
            scatter_done(i)

    run_pass(table1)
    plsc.subcore_barrier()

    b1v = bbuf[...]

    def load_rows(nrows):
        pltpu.sync_copy(acc.at[pl.ds(nbase, nrows)], ybuf.at[pl.ds(0, nrows)])
        pltpu.sync_copy(table1.at[pl.ds(nbase, nrows)],
                        xbuf.at[pl.ds(0, nrows)])

    @pl.when(s < last)
    def _():
        load_rows(TROWS_PT)

    @pl.when(s == last)
    def _():
        load_rows(TROWS_LAST)

    def midrow(k, carry):
        r = r0 + 2 * k
        yv = plsc.load_gather(ybuf, [r, cidx])
        xv = plsc.load_gather(xbuf, [r, cidx])
        dv = plsc.load_gather(disb, [r])
        hv = jnp.maximum(dv * (yv + xv) + b1v, 0.0)
        plsc.store_scatter(t2buf, [r, cidx], hv * dv)
        return carry

    lax.fori_loop(0, npairs, midrow, 0)

    def put_t2(nrows, arows):
        pltpu.sync_copy(t2buf.at[pl.ds(0, nrows)],
                        table2.at[pl.ds(nbase, nrows)])
        pltpu.sync_copy(zeros8_hbm.at[pl.ds(abase, arows)],
                        acc.at[pl.ds(abase, arows)])

    @pl.when(s < last)
    def _():
        put_t2(TROWS_PT, ROWS_PT)

    @pl.when(s == last)
    def _():
        put_t2(TROWS_LAST, ROWS_LAST)

    plsc.subcore_barrier()
    run_pass(table2)
    plsc.subcore_barrier()

    @pl.when(s < last)
    def _():
        pltpu.sync_copy(acc.at[pl.ds(nbase, TROWS_PT)],
                        ybuf.at[pl.ds(0, TROWS_PT)])

    @pl.when(s == last)
    def _():
        pltpu.sync_copy(acc.at[pl.ds(nbase, TROWS_LAST)],
                        ybuf.at[pl.ds(0, TROWS_LAST)])

    def finrow(k, carry):
        r = r0 + 2 * k
        yv = plsc.load_gather(ybuf, [r, cidx])
        xv = plsc.load_gather(t2buf, [r, cidx])
        dv = plsc.load_gather(disb, [r])
        plsc.store_scatter(gbuf, [r, cidx], dv * (yv + xv))
        return carry

    lax.fori_loop(0, npairs, finrow, 0)

    def put_g(nrows):
        for cc in range(NC):
            @pl.when(c == cc)
            def _():
                pltpu.sync_copy(
                    gbuf.at[pl.ds(0, nrows)],
                    out_hbm.at[pl.ds(nbase, nrows), pl.ds(cc * HH, HH)])

    @pl.when(s < last)
    def _():
        put_g(TROWS_PT)

    @pl.when(s == last)
    def _():
        put_g(TROWS_LAST)


def _tc_mm_body(x_ref, w1_ref, xw_ref):
    xw_ref[...] = jnp.dot(x_ref[...], w1_ref[...],
                          preferred_element_type=jnp.float32)


def _tc3_body(g_ref, w2_ref, b2_ref, out_ref):
    o = jnp.dot(g_ref[...], w2_ref[...],
                preferred_element_type=jnp.float32) + b2_ref[...]
    m = jnp.max(o, axis=1, keepdims=True)
    e = o - m
    lse = jnp.log(jnp.sum(jnp.exp(e), axis=1, keepdims=True))
    out_ref[...] = e - lse


def kernel(x, edge_index, W1, b1, W2, b2):
    e32 = edge_index.astype(jnp.int32)
    padcols = jnp.stack([
        jnp.zeros((EPAD - E,), jnp.int32),
        jnp.full((EPAD - E,), DISCARD, jnp.int32),
    ])
    edges = jnp.concatenate([e32, padcols], axis=1).reshape(2, NCHUNK, CHUNK)
    z1 = jnp.zeros((NPAD,), jnp.float32)
    z8 = jnp.zeros((NPAD, HH), jnp.float32)
    b1p = jnp.stack([jnp.tile(b1[:HH], 2), jnp.tile(b1[HH:], 2)])

    xw = pl.pallas_call(
        _tc_mm_body,
        out_shape=jax.ShapeDtypeStruct((N, HID), jnp.float32),
    )(x, W1)

    g = _sc_mega(xw, edges, b1p, z8, z1)

    out = pl.pallas_call(
        _tc3_body,
        out_shape=jax.ShapeDtypeStruct((N, OUT_CH), jnp.float32),
    )(g, W2, b2.reshape(1, OUT_CH))
    return out


# speedup vs baseline: 1.2673x; 1.0150x over previous
import functools

import jax
import jax.numpy as jnp
from jax import lax
from jax.experimental import pallas as pl
from jax.experimental.pallas import tpu as pltpu
from jax.experimental.pallas import tpu_sc as plsc

N = 10000
NPAD = 10016
DISCARD = 10008
IN_CH = 128
HID = 16
HH = 8
OUT_CH = 40
E = 320000
NC = 2
NS = 16
NW = NC * NS
CHUNK = 128
NCHUNK = 2560
EPAD = NCHUNK * CHUNK
CPTM = NCHUNK // NS
NBUF = 8
ROWS_PT = 632
ROWS_LAST = NPAD - ROWS_PT * (NS - 1)
TROWS_PT = 632
TROWS_LAST = N - TROWS_PT * (NS - 1)

_MESH = plsc.VectorSubcoreMesh(
    core_axis_name="c", subcore_axis_name="s", num_cores=NC, num_subcores=NS
)
_SC_PARAMS_NL = pltpu.CompilerParams(use_tc_tiling_on_sc=False,
                                     needs_layout_passes=False)


@functools.partial(
    pl.kernel,
    out_type=jax.ShapeDtypeStruct((N, HID), jnp.float32),
    mesh=_MESH,
    scratch_types=[
        pltpu.VMEM((CPTM, CHUNK), jnp.int32),
        pltpu.VMEM((CPTM, CHUNK), jnp.int32),
        [pltpu.VMEM((CHUNK, HH), jnp.float32)] * NBUF,
        [pltpu.SemaphoreType.DMA] * NBUF,
        [pltpu.SemaphoreType.DMA] * NBUF,
        pltpu.VMEM((TROWS_PT, HH), jnp.float32),
        pltpu.VMEM((TROWS_PT, HH), jnp.float32),
        pltpu.VMEM((TROWS_PT, HH), jnp.float32),
        pltpu.VMEM((TROWS_PT, HH), jnp.float32),
        pltpu.VMEM((16,), jnp.float32),
        pltpu.VMEM((CHUNK,), jnp.float32),
        pltpu.VMEM((640,), jnp.float32),
        pltpu.SemaphoreType.DMA,
        pltpu.VMEM_SHARED((N, HH), jnp.float32),
        pltpu.VMEM_SHARED((N, HH), jnp.float32),
        pltpu.VMEM_SHARED((NPAD, HH), jnp.float32),
        pltpu.VMEM_SHARED((NPAD,), jnp.float32),
    ],
    compiler_params=_SC_PARAMS_NL,
)
def _sc_mega(xw_hbm, edges_hbm, b1p_hbm, zeros8_hbm, zeros1_hbm, out_hbm,
             src_v, dst_v, rows, gsem, ssem,
             ybuf, xbuf, t2buf, gbuf, bbuf, ones_v, disb, dsem,
             table1, table2, acc, dacc):
    c = lax.axis_index("c")
    s = lax.axis_index("s")
    start = pl.multiple_of(CPTM * s, 8)
    pltpu.sync_copy(edges_hbm.at[0, pl.ds(start, CPTM)], src_v)
    pltpu.sync_copy(edges_hbm.at[1, pl.ds(start, CPTM)], dst_v)
    pltpu.sync_copy(b1p_hbm.at[c], bbuf)
    for i in range(CHUNK // 16):
        ones_v[pl.ds(i * 16, 16)] = jnp.ones((16,), jnp.float32)

    nbase = pl.multiple_of(s * TROWS_PT, 8)
    abase = pl.multiple_of(s * ROWS_PT, 8)
    last = NS - 1

    @pl.when(s < last)
    def _():
        pltpu.sync_copy(zeros1_hbm.at[pl.ds(abase, ROWS_PT)],
                        dacc.at[pl.ds(abase, ROWS_PT)])

    @pl.when(s == last)
    def _():
        pltpu.sync_copy(zeros1_hbm.at[pl.ds(abase, ROWS_LAST)],
                        dacc.at[pl.ds(abase, ROWS_LAST)])

    plsc.subcore_barrier()

    def dfire(j, carry):
        pltpu.async_copy(ones_v, dacc.at[dst_v.at[j]], dsem, add=True)
        return carry

    lax.fori_loop(0, CPTM, dfire, 0)

    def stage(nrows, arows):
        for cc in range(NC):
            @pl.when(c == cc)
            def _():
                pltpu.sync_copy(
                    xw_hbm.at[pl.ds(nbase, nrows), pl.ds(cc * HH, HH)],
                    xbuf.at[pl.ds(0, nrows)])
        pltpu.sync_copy(zeros8_hbm.at[pl.ds(abase, arows)],
                        acc.at[pl.ds(abase, arows)])

    @pl.when(s < last)
    def _():
        stage(TROWS_PT, ROWS_PT)

    @pl.when(s == last)
    def _():
        stage(TROWS_LAST, ROWS_LAST)

    def ddrain(j, carry):
        pltpu.make_async_copy(ones_v, dacc.at[dst_v.at[0]], dsem).wait()
        return carry

    lax.fori_loop(0, CPTM, ddrain, 0)
    plsc.subcore_barrier()

    @pl.when(s < last)
    def _():
        pltpu.sync_copy(dacc.at[pl.ds(nbase, TROWS_PT)],
                        disb.at[pl.ds(0, TROWS_PT)])

    @pl.when(s == last)
    def _():
        pltpu.sync_copy(dacc.at[pl.ds(nbase, TROWS_LAST)],
                        disb.at[pl.ds(0, TROWS_LAST)])

    magic = jnp.full((16,), 0x5F3759DF, jnp.int32)

    def newton(k, carry):
        xv = disb[pl.ds(k * 16, 16)] + 1.0
        iv = magic - lax.shift_right_logical(plsc.bitcast(xv, jnp.int32), 1)
        yv = plsc.bitcast(iv, jnp.float32)
        for _ in range(3):
            yv = yv * (1.5 - 0.5 * xv * yv * yv)
        disb[pl.ds(k * 16, 16)] = yv
        return carry

    lax.fori_loop(0, 640 // 16, newton, 0)

    i16 = lax.iota(jnp.int32, 16)
    r0 = lax.shift_right_logical(i16, 3)
    cidx = lax.bitwise_and(i16, jnp.full((16,), 7, jnp.int32))
    npairs = jnp.where(s == last, TROWS_LAST // 2, TROWS_PT // 2)

    def prerow(k, carry):
        r = r0 + 2 * k
        dv = plsc.load_gather(disb, [r])
        tv = plsc.load_gather(xbuf, [r, cidx]) * dv
        plsc.store_scatter(gbuf, [r, cidx], tv)
        return carry

    lax.fori_loop(0, npairs, prerow, 0)

    @pl.when(s < last)
    def _():
        pltpu.sync_copy(gbuf.at[pl.ds(0, TROWS_PT)],
                        table1.at[pl.ds(nbase, TROWS_PT)])

    @pl.when(s == last)
    def _():
        pltpu.sync_copy(gbuf.at[pl.ds(0, TROWS_LAST)],
                        table1.at[pl.ds(nbase, TROWS_LAST)])

    plsc.subcore_barrier()

    def run_pass(table):
        def gather(k, i):
            pltpu.async_copy(table.at[src_v.at[k]], rows[i], gsem[i])

        def gather_done(i):
            pltpu.make_async_copy(table.at[src_v.at[0]], rows[i],
                                  gsem[i]).wait()

        def scatter(k, i):
            pltpu.async_copy(rows[i], acc.at[dst_v.at[k]], ssem[i], add=True)

        def scatter_done(i):
            pltpu.make_async_copy(rows[i], acc.at[dst_v.at[0]],
                                  ssem[i]).wait()

        for i in range(NBUF):
            gather(i, i)

        nsup = CPTM // NBUF

        def superchunk(t, carry):
            base = NBUF * t
            for i in range(NBUF):
                k = base + i
                gather_done(i)
                scatter(k, i)

                @pl.when(t < nsup - 1)
                def _():
                    scatter_done(i)
                    gather(k + NBUF, i)
            return carry

        lax.fori_loop(0, nsup, superchunk, 0)
        for i in range(NBUF):
            scatter_done(i)

    run_pass(table1)
    plsc.subcore_barrier()

    b1v = bbuf[...]

    def load_rows(nrows):
        pltpu.sync_copy(acc.at[pl.ds(nbase, nrows)], ybuf.at[pl.ds(0, nrows)])
        pltpu.sync_copy(table1.at[pl.ds(nbase, nrows)],
                        xbuf.at[pl.ds(0, nrows)])

    @pl.when(s < last)
    def _():
        load_rows(TROWS_PT)

    @pl.when(s == last)
    def _():
        load_rows(TROWS_LAST)

    def midrow(k, carry):
        r = r0 + 2 * k
        yv = plsc.load_gather(ybuf, [r, cidx])
        xv = plsc.load_gather(xbuf, [r, cidx])
        dv = plsc.load_gather(disb, [r])
        hv = jnp.maximum(dv * (yv + xv) + b1v, 0.0)
        plsc.store_scatter(t2buf, [r, cidx], hv * dv)
        return carry

    lax.fori_loop(0, npairs, midrow, 0)

    def put_t2(nrows, arows):
        pltpu.sync_copy(t2buf.at[pl.ds(0, nrows)],
                        table2.at[pl.ds(nbase, nrows)])
        pltpu.sync_copy(zeros8_hbm.at[pl.ds(abase, arows)],
                        acc.at[pl.ds(abase, arows)])

    @pl.when(s < last)
    def _():
        put_t2(TROWS_PT, ROWS_PT)

    @pl.when(s == last)
    def _():
        put_t2(TROWS_LAST, ROWS_LAST)

    plsc.subcore_barrier()
    run_pass(table2)
    plsc.subcore_barrier()

    @pl.when(s < last)
    def _():
        pltpu.sync_copy(acc.at[pl.ds(nbase, TROWS_PT)],
                        ybuf.at[pl.ds(0, TROWS_PT)])

    @pl.when(s == last)
    def _():
        pltpu.sync_copy(acc.at[pl.ds(nbase, TROWS_LAST)],
                        ybuf.at[pl.ds(0, TROWS_LAST)])

    def finrow(k, carry):
        r = r0 + 2 * k
        yv = plsc.load_gather(ybuf, [r, cidx])
        xv = plsc.load_gather(t2buf, [r, cidx])
        dv = plsc.load_gather(disb, [r])
        plsc.store_scatter(gbuf, [r, cidx], dv * (yv + xv))
        return carry

    lax.fori_loop(0, npairs, finrow, 0)

    def put_g(nrows):
        for cc in range(NC):
            @pl.when(c == cc)
            def _():
                pltpu.sync_copy(
                    gbuf.at[pl.ds(0, nrows)],
                    out_hbm.at[pl.ds(nbase, nrows), pl.ds(cc * HH, HH)])

    @pl.when(s < last)
    def _():
        put_g(TROWS_PT)

    @pl.when(s == last)
    def _():
        put_g(TROWS_LAST)


def _tc_mm_body(x_ref, w1_ref, xw_ref):
    xw_ref[...] = jnp.dot(x_ref[...], w1_ref[...],
                          preferred_element_type=jnp.float32)


def _tc3_body(g_ref, w2_ref, b2_ref, out_ref):
    o = jnp.dot(g_ref[...], w2_ref[...],
                preferred_element_type=jnp.float32) + b2_ref[...]
    m = jnp.max(o, axis=1, keepdims=True)
    e = o - m
    lse = jnp.log(jnp.sum(jnp.exp(e), axis=1, keepdims=True))
    out_ref[...] = e - lse


def kernel(x, edge_index, W1, b1, W2, b2):
    e32 = edge_index.astype(jnp.int32)
    padcols = jnp.stack([
        jnp.zeros((EPAD - E,), jnp.int32),
        jnp.full((EPAD - E,), DISCARD, jnp.int32),
    ])
    edges = jnp.concatenate([e32, padcols], axis=1).reshape(2, NCHUNK, CHUNK)
    z1 = jnp.zeros((NPAD,), jnp.float32)
    z8 = jnp.zeros((NPAD, HH), jnp.float32)
    b1p = jnp.stack([jnp.tile(b1[:HH], 2), jnp.tile(b1[HH:], 2)])

    xw = pl.pallas_call(
        _tc_mm_body,
        out_shape=jax.ShapeDtypeStruct((N, HID), jnp.float32),
    )(x, W1)

    g = _sc_mega(xw, edges, b1p, z8, z1)

    out = pl.pallas_call(
        _tc3_body,
        out_shape=jax.ShapeDtypeStruct((N, OUT_CH), jnp.float32),
    )(g, W2, b2.reshape(1, OUT_CH))
    return out
